# baseline (speedup 1.0000x reference)
import jax
import jax.numpy as jnp
from jax import lax
from jax.experimental import pallas as pl
from jax.experimental.pallas import tpu as pltpu

_LANES = 128
_VALID_LANES = 96


def _bitonic_sort_kernel(x_ref, o_ref):
    x = x_ref[0, 0]
    rows = x.shape[0]
    row_log = rows.bit_length() - 1
    n_log = row_log + 7

    row_iota = lax.broadcasted_iota(jnp.int32, x.shape, 0)
    lane_iota = lax.broadcasted_iota(jnp.int32, x.shape, 1)

    def split_bits(v_log):
        is_row = v_log < row_log
        vr = jnp.where(is_row, jnp.left_shift(1, v_log), 0)
        vl = jnp.where(is_row, 0,
                       jnp.left_shift(1, jnp.maximum(v_log - row_log, 0)))
        return vr, vl

    def stage(x, k_log, j_log):
        jr, jl = split_bits(j_log)
        kr, kl = split_bits(k_log)
        is_lo = ((row_iota & jr) | (lane_iota & jl)) == 0
        up = ((row_iota & kr) | (lane_iota & kl)) == 0

        def cmpx(x, axis, size, dist):
            y = pltpu.roll(x, size - dist, axis)
            m = jnp.minimum(x, y)
            mx = jnp.maximum(x, y)
            w = jnp.where(up, mx, m)
            back = pltpu.roll(w, dist, axis)
            return jnp.where(is_lo, jnp.where(up, m, mx), back)

        return lax.cond(
            j_log < row_log,
            lambda x: cmpx(x, 0, rows, jr),
            lambda x: cmpx(x, 1, _LANES, jl),
            x,
        )

    def level(k_log, x):
        def one_pass(t, x):
            return stage(x, k_log, k_log - 1 - t)
        return lax.fori_loop(0, k_log, one_pass, x)

    x = lax.fori_loop(1, n_log + 1, level, x)
    o_ref[0, 0] = x


def _diff_norm_kernel(s_ref, o_ref):
    a = s_ref[0, 0]
    b = s_ref[0, 1]
    lane_iota = lax.broadcasted_iota(jnp.int32, a.shape, 1)
    d = jnp.where(lane_iota < _VALID_LANES, a - b, 0.0)
    o_ref[0] = jnp.full(o_ref.shape[1:], jnp.sum(d * d), jnp.float32)


def kernel(pc1, pc2):
    B = pc1.shape[0]
    n = pc1.shape[1] * pc1.shape[2]
    rows = n // _VALID_LANES
    assert rows * _VALID_LANES == n and rows & (rows - 1) == 0

    def prep(pc):
        v = pc.reshape(B, _VALID_LANES, rows).transpose(0, 2, 1)
        return jnp.pad(v, ((0, 0), (0, 0), (0, _LANES - _VALID_LANES)),
                       constant_values=jnp.inf)

    x = jnp.stack([prep(pc1), prep(pc2)], axis=1)

    sorted_x = pl.pallas_call(
        _bitonic_sort_kernel,
        grid=(B, 2),
        in_specs=[pl.BlockSpec((1, 1, rows, _LANES), lambda i, j: (i, j, 0, 0))],
        out_specs=pl.BlockSpec((1, 1, rows, _LANES), lambda i, j: (i, j, 0, 0)),
        out_shape=jax.ShapeDtypeStruct((B, 2, rows, _LANES), jnp.float32),
        compiler_params=pltpu.CompilerParams(
            dimension_semantics=("parallel", "parallel")),
    )(x)

    ss = pl.pallas_call(
        _diff_norm_kernel,
        grid=(B,),
        in_specs=[pl.BlockSpec((1, 2, rows, _LANES), lambda i: (i, 0, 0, 0))],
        out_specs=pl.BlockSpec((1, 8, _LANES), lambda i: (i, 0, 0)),
        out_shape=jax.ShapeDtypeStruct((B, 8, _LANES), jnp.float32),
        compiler_params=pltpu.CompilerParams(
            dimension_semantics=("parallel",)),
    )(sorted_x)

    return jnp.mean(jnp.sqrt(ss[:, 0, 0]))


# speedup vs baseline: 1.2079x; 1.2079x over previous
import jax
import jax.numpy as jnp
from jax import lax
from jax.experimental import pallas as pl
from jax.experimental.pallas import tpu as pltpu

_LANES = 128
_VALID_LANES = 96


def _bitonic_sort_kernel(x_ref, o_ref):
    x = x_ref[0, 0]
    rows = x.shape[0]
    row_log = rows.bit_length() - 1
    n_log = row_log + 7

    row_iota = lax.broadcasted_iota(jnp.int32, x.shape, 0)
    lane_iota = lax.broadcasted_iota(jnp.int32, x.shape, 1)

    def split_bits(v_log):
        is_row = v_log < row_log
        vr = jnp.where(is_row, jnp.left_shift(1, v_log), 0)
        vl = jnp.where(is_row, 0,
                       jnp.left_shift(1, jnp.maximum(v_log - row_log, 0)))
        return vr, vl

    def cmpx(x, up, is_lo, axis, size, dist):
        y = pltpu.roll(x, size - dist, axis)
        m = jnp.minimum(x, y)
        mx = jnp.maximum(x, y)
        w = jnp.where(up, mx, m)
        back = pltpu.roll(w, dist, axis)
        return jnp.where(is_lo, jnp.where(up, m, mx), back)

    def level(k_log, x):
        kr, kl = split_bits(k_log)
        up = ((row_iota & kr) | (lane_iota & kl)) == 0

        def lane_pass(t, x):
            dl = jnp.left_shift(1, k_log - 1 - row_log - t)
            is_lo = (lane_iota & dl) == 0
            return cmpx(x, up, is_lo, 1, _LANES, dl)

        def row_pass(t, x):
            dr = jnp.left_shift(1, jnp.minimum(k_log, row_log) - 1 - t)
            is_lo = (row_iota & dr) == 0
            return cmpx(x, up, is_lo, 0, rows, dr)

        x = lax.fori_loop(0, jnp.maximum(k_log - row_log, 0), lane_pass, x)
        return lax.fori_loop(0, jnp.minimum(k_log, row_log), row_pass, x)

    x = lax.fori_loop(1, n_log + 1, level, x)
    o_ref[0, 0] = x


def _diff_norm_kernel(s_ref, o_ref):
    a = s_ref[0, 0]
    b = s_ref[0, 1]
    lane_iota = lax.broadcasted_iota(jnp.int32, a.shape, 1)
    d = jnp.where(lane_iota < _VALID_LANES, a - b, 0.0)
    o_ref[0] = jnp.full(o_ref.shape[1:], jnp.sum(d * d), jnp.float32)


def kernel(pc1, pc2):
    B = pc1.shape[0]
    n = pc1.shape[1] * pc1.shape[2]
    rows = n // _VALID_LANES
    assert rows * _VALID_LANES == n and rows & (rows - 1) == 0

    def prep(pc):
        v = pc.reshape(B, _VALID_LANES, rows).transpose(0, 2, 1)
        return jnp.pad(v, ((0, 0), (0, 0), (0, _LANES - _VALID_LANES)),
                       constant_values=jnp.inf)

    x = jnp.stack([prep(pc1), prep(pc2)], axis=1)

    sorted_x = pl.pallas_call(
        _bitonic_sort_kernel,
        grid=(B, 2),
        in_specs=[pl.BlockSpec((1, 1, rows, _LANES), lambda i, j: (i, j, 0, 0))],
        out_specs=pl.BlockSpec((1, 1, rows, _LANES), lambda i, j: (i, j, 0, 0)),
        out_shape=jax.ShapeDtypeStruct((B, 2, rows, _LANES), jnp.float32),
        compiler_params=pltpu.CompilerParams(
            dimension_semantics=("parallel", "parallel")),
    )(x)

    ss = pl.pallas_call(
        _diff_norm_kernel,
        grid=(B,),
        in_specs=[pl.BlockSpec((1, 2, rows, _LANES), lambda i: (i, 0, 0, 0))],
        out_specs=pl.BlockSpec((1, 8, _LANES), lambda i: (i, 0, 0)),
        out_shape=jax.ShapeDtypeStruct((B, 8, _LANES), jnp.float32),
        compiler_params=pltpu.CompilerParams(
            dimension_semantics=("parallel",)),
    )(sorted_x)

    return jnp.mean(jnp.sqrt(ss[:, 0, 0]))


# speedup vs baseline: 5.2101x; 4.3132x over previous
import jax
import jax.numpy as jnp
from jax import lax
from jax.experimental import pallas as pl
from jax.experimental.pallas import tpu as pltpu

_LANES = 128
_VALID_LANES = 96


def _bitonic_sort_kernel(x_ref, o_ref):
    rows = x_ref.shape[2]
    row_log = rows.bit_length() - 1
    n_log = row_log + 7
    chunk_log = min(6, row_log)
    C = 1 << chunk_log
    nchunks = rows // C

    riota = lax.broadcasted_iota(jnp.int32, (C, _LANES), 0)
    liota = lax.broadcasted_iota(jnp.int32, (C, _LANES), 1)

    def up_mask(k_log, c1):
        if k_log < chunk_log:
            return (riota & (1 << k_log)) == 0
        if k_log < row_log:
            return (c1 & (1 << (k_log - chunk_log))) == 0
        return (liota & (1 << (k_log - row_log))) == 0

    def cx_routed(x, up, dr):
        g = x.reshape(C // (2 * dr), 2, dr, _LANES)
        a, b = g[:, 0], g[:, 1]
        m = jnp.minimum(a, b)
        mx = jnp.maximum(a, b)
        if isinstance(up, jax.Array) and up.ndim:
            up = up.reshape(C // (2 * dr), 2, dr, _LANES)[:, 0]
        lo = jnp.where(up, m, mx)
        hi = jnp.where(up, mx, m)
        return jnp.stack([lo, hi], axis=1).reshape(C, _LANES)

    def cx_roll(x, up, axis, size, d, iota):
        is_lo = (iota & d) == 0
        y = pltpu.roll(x, size - d, axis)
        z = pltpu.roll(x, d, axis)
        p = jnp.where(is_lo, y, z)
        m = jnp.minimum(x, p)
        mx = jnp.maximum(x, p)
        return jnp.where(up == is_lo, m, mx)

    def row_chain(x, k_log, c1, start_j):
        up = up_mask(k_log, c1)
        for j in range(start_j, -1, -1):
            d = 1 << j
            if d >= 8:
                x = cx_routed(x, up, d)
            else:
                x = cx_roll(x, up, 0, C, d, riota)
        return x

    def lane_chain(x, k_log, lane_js):
        up = up_mask(k_log, 0)
        for j in lane_js:
            x = cx_roll(x, up, 1, _LANES, 1 << (j - row_log), liota)
        return x

    def rd(c):
        return o_ref[0, 0, pl.ds(c * C, C), :]

    def wr(c, v):
        o_ref[0, 0, pl.ds(c * C, C), :] = v

    def sweep1(c, _):
        x = x_ref[0, 0, pl.ds(c * C, C), :]
        for k_log in range(1, chunk_log + 1):
            x = row_chain(x, k_log, c, k_log - 1)
        wr(c, x)
        return 0

    lax.fori_loop(0, nchunks, sweep1, 0)

    for k_log in range(chunk_log + 1, n_log + 1):
        lane_js = list(range(k_log - 1, row_log - 1, -1))
        cross_js = list(range(min(k_log - 1, row_log - 1), chunk_log - 1, -1))

        if not cross_js:
            def solo_sweep(c, _, k_log=k_log, lane_js=lane_js):
                x = rd(c)
                x = lane_chain(x, k_log, lane_js)
                x = row_chain(x, k_log, c, chunk_log - 1)
                wr(c, x)
                return 0

            lax.fori_loop(0, nchunks, solo_sweep, 0)
            continue

        def cross_pairs(j, body_fn):
            s = 1 << (j - chunk_log)
            b = j - chunk_log

            def body(g, _):
                c1 = ((g >> b) << (b + 1)) | (g & (s - 1))
                body_fn(c1, c1 + s)
                return 0

            lax.fori_loop(0, nchunks // 2, body, 0)

        def cross_cx(c1, c2, a, bv, k_log=k_log):
            up = up_mask(k_log, c1)
            m = jnp.minimum(a, bv)
            mx = jnp.maximum(a, bv)
            return jnp.where(up, m, mx), jnp.where(up, mx, m)

        def first_sweep(c1, c2, k_log=k_log, lane_js=lane_js):
            a, bv = rd(c1), rd(c2)
            a = lane_chain(a, k_log, lane_js)
            bv = lane_chain(bv, k_log, lane_js)
            a, bv = cross_cx(c1, c2, a, bv)
            wr(c1, a)
            wr(c2, bv)

        cross_pairs(cross_js[0], first_sweep)

        for j in cross_js[1:-1]:
            def mid_sweep(c1, c2, k_log=k_log):
                a, bv = cross_cx(c1, c2, rd(c1), rd(c2))
                wr(c1, a)
                wr(c2, bv)

            cross_pairs(j, mid_sweep)

        if len(cross_js) > 1:
            def last_sweep(c1, c2, k_log=k_log):
                a, bv = cross_cx(c1, c2, rd(c1), rd(c2))
                a = row_chain(a, k_log, c1, chunk_log - 1)
                bv = row_chain(bv, k_log, c2, chunk_log - 1)
                wr(c1, a)
                wr(c2, bv)

            cross_pairs(chunk_log, last_sweep)
        else:
            def tail_sweep(c, _, k_log=k_log):
                wr(c, row_chain(rd(c), k_log, c, chunk_log - 1))
                return 0

            lax.fori_loop(0, nchunks, tail_sweep, 0)


def _diff_norm_kernel(s_ref, o_ref):
    a = s_ref[0, 0]
    b = s_ref[0, 1]
    lane_iota = lax.broadcasted_iota(jnp.int32, a.shape, 1)
    d = jnp.where(lane_iota < _VALID_LANES, a - b, 0.0)
    o_ref[0] = jnp.full(o_ref.shape[1:], jnp.sum(d * d), jnp.float32)


def kernel(pc1, pc2):
    B = pc1.shape[0]
    n = pc1.shape[1] * pc1.shape[2]
    rows = n // _VALID_LANES
    assert rows * _VALID_LANES == n and rows & (rows - 1) == 0

    def prep(pc):
        v = pc.reshape(B, _VALID_LANES, rows).transpose(0, 2, 1)
        return jnp.pad(v, ((0, 0), (0, 0), (0, _LANES - _VALID_LANES)),
                       constant_values=jnp.inf)

    x = jnp.stack([prep(pc1), prep(pc2)], axis=1)

    sorted_x = pl.pallas_call(
        _bitonic_sort_kernel,
        grid=(B, 2),
        in_specs=[pl.BlockSpec((1, 1, rows, _LANES), lambda i, j: (i, j, 0, 0))],
        out_specs=pl.BlockSpec((1, 1, rows, _LANES), lambda i, j: (i, j, 0, 0)),
        out_shape=jax.ShapeDtypeStruct((B, 2, rows, _LANES), jnp.float32),
        compiler_params=pltpu.CompilerParams(
            dimension_semantics=("parallel", "parallel")),
    )(x)

    ss = pl.pallas_call(
        _diff_norm_kernel,
        grid=(B,),
        in_specs=[pl.BlockSpec((1, 2, rows, _LANES), lambda i: (i, 0, 0, 0))],
        out_specs=pl.BlockSpec((1, 8, _LANES), lambda i: (i, 0, 0)),
        out_shape=jax.ShapeDtypeStruct((B, 8, _LANES), jnp.float32),
        compiler_params=pltpu.CompilerParams(
            dimension_semantics=("parallel",)),
    )(sorted_x)

    return jnp.mean(jnp.sqrt(ss[:, 0, 0]))


# speedup vs baseline: 5.2586x; 1.0093x over previous
import jax
import jax.numpy as jnp
from jax import lax
from jax.experimental import pallas as pl
from jax.experimental.pallas import tpu as pltpu

_LANES = 128
_VALID_LANES = 96


def _bitonic_sort_kernel(x_ref, o_ref):
    rows = x_ref.shape[2]
    row_log = rows.bit_length() - 1
    n_log = row_log + 7
    chunk_log = min(6, row_log)
    C = 1 << chunk_log
    nchunks = rows // C

    riota = lax.broadcasted_iota(jnp.int32, (C, _LANES), 0)
    liota = lax.broadcasted_iota(jnp.int32, (C, _LANES), 1)

    def up_mask(k_log, c1):
        if k_log < chunk_log:
            return (riota & (1 << k_log)) == 0
        if k_log < row_log:
            return (c1 & (1 << (k_log - chunk_log))) == 0
        return (liota & (1 << (k_log - row_log))) == 0

    def cx_routed(x, up, dr):
        g = x.reshape(C // (2 * dr), 2, dr, _LANES)
        a, b = g[:, 0], g[:, 1]
        m = jnp.minimum(a, b)
        mx = jnp.maximum(a, b)
        if isinstance(up, jax.Array) and up.ndim:
            up = up.reshape(C // (2 * dr), 2, dr, _LANES)[:, 0]
        lo = jnp.where(up, m, mx)
        hi = jnp.where(up, mx, m)
        return jnp.stack([lo, hi], axis=1).reshape(C, _LANES)

    def cx_roll(x, up, axis, size, d, iota):
        is_lo = (iota & d) == 0
        y = pltpu.roll(x, size - d, axis)
        z = pltpu.roll(x, d, axis)
        p = jnp.where(is_lo, y, z)
        m = jnp.minimum(x, p)
        mx = jnp.maximum(x, p)
        return jnp.where(up == is_lo, m, mx)

    def row_chain(x, k_log, c1, start_j):
        up = up_mask(k_log, c1)
        for j in range(start_j, -1, -1):
            d = 1 << j
            if d >= 8:
                x = cx_routed(x, up, d)
            else:
                x = cx_roll(x, up, 0, C, d, riota)
        return x

    def lane_chain(x, k_log, lane_js):
        up = up_mask(k_log, 0)
        for j in lane_js:
            x = cx_roll(x, up, 1, _LANES, 1 << (j - row_log), liota)
        return x

    def rd(c):
        return o_ref[0, 0, pl.ds(c * C, C), :]

    def wr(c, v):
        o_ref[0, 0, pl.ds(c * C, C), :] = v

    def sweep1_one(c):
        x = x_ref[0, 0, pl.ds(c * C, C), :]
        for k_log in range(1, chunk_log + 1):
            x = row_chain(x, k_log, c, k_log - 1)
        wr(c, x)

    if nchunks % 2 == 0:
        def sweep1(g, _):
            sweep1_one(2 * g)
            sweep1_one(2 * g + 1)
            return 0

        lax.fori_loop(0, nchunks // 2, sweep1, 0)
    else:
        def sweep1(c, _):
            sweep1_one(c)
            return 0

        lax.fori_loop(0, nchunks, sweep1, 0)

    for k_log in range(chunk_log + 1, n_log + 1):
        lane_js = list(range(k_log - 1, row_log - 1, -1))
        cross_js = list(range(min(k_log - 1, row_log - 1), chunk_log - 1, -1))

        if not cross_js:
            def solo_sweep(c, _, k_log=k_log, lane_js=lane_js):
                x = rd(c)
                x = lane_chain(x, k_log, lane_js)
                x = row_chain(x, k_log, c, chunk_log - 1)
                wr(c, x)
                return 0

            lax.fori_loop(0, nchunks, solo_sweep, 0)
            continue

        def cross_pairs(j, body_fn):
            s = 1 << (j - chunk_log)
            b = j - chunk_log

            def body(g, _):
                c1 = ((g >> b) << (b + 1)) | (g & (s - 1))
                body_fn(c1, c1 + s)
                return 0

            lax.fori_loop(0, nchunks // 2, body, 0)

        def cross_cx(c1, c2, a, bv, k_log=k_log):
            up = up_mask(k_log, c1)
            m = jnp.minimum(a, bv)
            mx = jnp.maximum(a, bv)
            return jnp.where(up, m, mx), jnp.where(up, mx, m)

        def first_sweep(c1, c2, k_log=k_log, lane_js=lane_js):
            a, bv = rd(c1), rd(c2)
            a = lane_chain(a, k_log, lane_js)
            bv = lane_chain(bv, k_log, lane_js)
            a, bv = cross_cx(c1, c2, a, bv)
            wr(c1, a)
            wr(c2, bv)

        cross_pairs(cross_js[0], first_sweep)

        for j in cross_js[1:-1]:
            def mid_sweep(c1, c2, k_log=k_log):
                a, bv = cross_cx(c1, c2, rd(c1), rd(c2))
                wr(c1, a)
                wr(c2, bv)

            cross_pairs(j, mid_sweep)

        if len(cross_js) > 1:
            def last_sweep(c1, c2, k_log=k_log):
                a, bv = cross_cx(c1, c2, rd(c1), rd(c2))
                a = row_chain(a, k_log, c1, chunk_log - 1)
                bv = row_chain(bv, k_log, c2, chunk_log - 1)
                wr(c1, a)
                wr(c2, bv)

            cross_pairs(chunk_log, last_sweep)
        else:
            def tail_sweep(c, _, k_log=k_log):
                wr(c, row_chain(rd(c), k_log, c, chunk_log - 1))
                return 0

            lax.fori_loop(0, nchunks, tail_sweep, 0)


def _diff_norm_kernel(s_ref, o_ref):
    a = s_ref[0, 0]
    b = s_ref[0, 1]
    lane_iota = lax.broadcasted_iota(jnp.int32, a.shape, 1)
    d = jnp.where(lane_iota < _VALID_LANES, a - b, 0.0)
    o_ref[0] = jnp.full(o_ref.shape[1:], jnp.sum(d * d), jnp.float32)


def kernel(pc1, pc2):
    B = pc1.shape[0]
    n = pc1.shape[1] * pc1.shape[2]
    rows = n // _VALID_LANES
    assert rows * _VALID_LANES == n and rows & (rows - 1) == 0

    def prep(pc):
        v = pc.reshape(B, _VALID_LANES, rows).transpose(0, 2, 1)
        return jnp.pad(v, ((0, 0), (0, 0), (0, _LANES - _VALID_LANES)),
                       constant_values=jnp.inf)

    x = jnp.stack([prep(pc1), prep(pc2)], axis=1)

    sorted_x = pl.pallas_call(
        _bitonic_sort_kernel,
        grid=(B, 2),
        in_specs=[pl.BlockSpec((1, 1, rows, _LANES), lambda i, j: (i, j, 0, 0))],
        out_specs=pl.BlockSpec((1, 1, rows, _LANES), lambda i, j: (i, j, 0, 0)),
        out_shape=jax.ShapeDtypeStruct((B, 2, rows, _LANES), jnp.float32),
        compiler_params=pltpu.CompilerParams(
            dimension_semantics=("parallel", "parallel")),
    )(x)

    ss = pl.pallas_call(
        _diff_norm_kernel,
        grid=(B,),
        in_specs=[pl.BlockSpec((1, 2, rows, _LANES), lambda i: (i, 0, 0, 0))],
        out_specs=pl.BlockSpec((1, 8, _LANES), lambda i: (i, 0, 0)),
        out_shape=jax.ShapeDtypeStruct((B, 8, _LANES), jnp.float32),
        compiler_params=pltpu.CompilerParams(
            dimension_semantics=("parallel",)),
    )(sorted_x)

    return jnp.mean(jnp.sqrt(ss[:, 0, 0]))


# speedup vs baseline: 5.3518x; 1.0177x over previous
import jax
import jax.numpy as jnp
from jax import lax
from jax.experimental import pallas as pl
from jax.experimental.pallas import tpu as pltpu

_LANES = 128
_VALID_LANES = 96


def _bitonic_sort_kernel(x_ref, o_ref):
    rows = x_ref.shape[2]
    row_log = rows.bit_length() - 1
    n_log = row_log + 7
    chunk_log = min(6, row_log)
    C = 1 << chunk_log
    nchunks = rows // C

    riota = lax.broadcasted_iota(jnp.int32, (C, _LANES), 0)
    liota = lax.broadcasted_iota(jnp.int32, (C, _LANES), 1)

    def up_mask(k_log, c1):
        if k_log < chunk_log:
            return (riota & (1 << k_log)) == 0
        if k_log < row_log:
            return (c1 & (1 << (k_log - chunk_log))) == 0
        return (liota & (1 << (k_log - row_log))) == 0

    def cx_routed(x, up, dr):
        g = x.reshape(C // (2 * dr), 2, dr, _LANES)
        a, b = g[:, 0], g[:, 1]
        m = jnp.minimum(a, b)
        mx = jnp.maximum(a, b)
        if isinstance(up, jax.Array) and up.ndim:
            up = up.reshape(C // (2 * dr), 2, dr, _LANES)[:, 0]
        lo = jnp.where(up, m, mx)
        hi = jnp.where(up, mx, m)
        return jnp.stack([lo, hi], axis=1).reshape(C, _LANES)

    def cx_roll(x, up, axis, size, d, iota):
        is_lo = (iota & d) == 0
        y = pltpu.roll(x, size - d, axis)
        z = pltpu.roll(x, d, axis)
        p = jnp.where(is_lo, y, z)
        m = jnp.minimum(x, p)
        mx = jnp.maximum(x, p)
        return jnp.where(up == is_lo, m, mx)

    def row_chain(x, k_log, c1, start_j):
        up = up_mask(k_log, c1)
        for j in range(start_j, -1, -1):
            d = 1 << j
            if d >= 8:
                x = cx_routed(x, up, d)
            else:
                x = cx_roll(x, up, 0, C, d, riota)
        return x

    def lane_chain(x, k_log, lane_js):
        up = up_mask(k_log, 0)
        for j in lane_js:
            x = cx_roll(x, up, 1, _LANES, 1 << (j - row_log), liota)
        return x

    def rd(c):
        return o_ref[0, 0, pl.ds(c * C, C), :]

    def wr(c, v):
        o_ref[0, 0, pl.ds(c * C, C), :] = v

    def sweep1_one(c):
        x = x_ref[0, 0, pl.ds(c * C, C), :]
        for k_log in range(1, chunk_log + 1):
            x = row_chain(x, k_log, c, k_log - 1)
        wr(c, x)

    if nchunks % 2 == 0:
        def sweep1(g, _):
            sweep1_one(2 * g)
            sweep1_one(2 * g + 1)
            return 0

        lax.fori_loop(0, nchunks // 2, sweep1, 0)
    else:
        def sweep1(c, _):
            sweep1_one(c)
            return 0

        lax.fori_loop(0, nchunks, sweep1, 0)

    for k_log in range(chunk_log + 1, n_log + 1):
        lane_js = list(range(k_log - 1, row_log - 1, -1))
        cross_js = list(range(min(k_log - 1, row_log - 1), chunk_log - 1, -1))

        if not cross_js:
            def solo_sweep(c, _, k_log=k_log, lane_js=lane_js):
                x = rd(c)
                x = lane_chain(x, k_log, lane_js)
                x = row_chain(x, k_log, c, chunk_log - 1)
                wr(c, x)
                return 0

            lax.fori_loop(0, nchunks, solo_sweep, 0)
            continue

        def cross_pairs(j, body_fn):
            s = 1 << (j - chunk_log)
            b = j - chunk_log

            def body(g, _):
                c1 = ((g >> b) << (b + 1)) | (g & (s - 1))
                body_fn(c1, c1 + s)
                return 0

            lax.fori_loop(0, nchunks // 2, body, 0)

        def cross_cx(c1, c2, a, bv, k_log=k_log):
            up = up_mask(k_log, c1)
            m = jnp.minimum(a, bv)
            mx = jnp.maximum(a, bv)
            return jnp.where(up, m, mx), jnp.where(up, mx, m)

        def first_sweep(c1, c2, k_log=k_log, lane_js=lane_js):
            a, bv = rd(c1), rd(c2)
            a = lane_chain(a, k_log, lane_js)
            bv = lane_chain(bv, k_log, lane_js)
            a, bv = cross_cx(c1, c2, a, bv)
            wr(c1, a)
            wr(c2, bv)

        cross_pairs(cross_js[0], first_sweep)

        mids = cross_js[1:-1]
        while mids:
            if len(mids) >= 2:
                j1 = mids[0]
                b = j1 - 1 - chunk_log
                s2 = 1 << b

                def quad_body(g, _, k_log=k_log, b=b, s2=s2):
                    c0 = ((g >> b) << (b + 2)) | (g & (s2 - 1))
                    va = rd(c0)
                    vb = rd(c0 + s2)
                    vc = rd(c0 + 2 * s2)
                    vd = rd(c0 + 3 * s2)
                    va, vc = cross_cx(c0, 0, va, vc)
                    vb, vd = cross_cx(c0, 0, vb, vd)
                    va, vb = cross_cx(c0, 0, va, vb)
                    vc, vd = cross_cx(c0, 0, vc, vd)
                    wr(c0, va)
                    wr(c0 + s2, vb)
                    wr(c0 + 2 * s2, vc)
                    wr(c0 + 3 * s2, vd)
                    return 0

                lax.fori_loop(0, nchunks // 4, quad_body, 0)
                mids = mids[2:]
            else:
                def mid_sweep(c1, c2, k_log=k_log):
                    a, bv = cross_cx(c1, c2, rd(c1), rd(c2))
                    wr(c1, a)
                    wr(c2, bv)

                cross_pairs(mids[0], mid_sweep)
                mids = mids[1:]

        if len(cross_js) > 1:
            def last_sweep(c1, c2, k_log=k_log):
                a, bv = cross_cx(c1, c2, rd(c1), rd(c2))
                a = row_chain(a, k_log, c1, chunk_log - 1)
                bv = row_chain(bv, k_log, c2, chunk_log - 1)
                wr(c1, a)
                wr(c2, bv)

            cross_pairs(chunk_log, last_sweep)
        else:
            def tail_sweep(c, _, k_log=k_log):
                wr(c, row_chain(rd(c), k_log, c, chunk_log - 1))
                return 0

            lax.fori_loop(0, nchunks, tail_sweep, 0)


def _diff_norm_kernel(s_ref, o_ref):
    a = s_ref[0, 0]
    b = s_ref[0, 1]
    lane_iota = lax.broadcasted_iota(jnp.int32, a.shape, 1)
    d = jnp.where(lane_iota < _VALID_LANES, a - b, 0.0)
    o_ref[0] = jnp.full(o_ref.shape[1:], jnp.sum(d * d), jnp.float32)


def kernel(pc1, pc2):
    B = pc1.shape[0]
    n = pc1.shape[1] * pc1.shape[2]
    rows = n // _VALID_LANES
    assert rows * _VALID_LANES == n and rows & (rows - 1) == 0

    def prep(pc):
        v = pc.reshape(B, _VALID_LANES, rows).transpose(0, 2, 1)
        return jnp.pad(v, ((0, 0), (0, 0), (0, _LANES - _VALID_LANES)),
                       constant_values=jnp.inf)

    x = jnp.stack([prep(pc1), prep(pc2)], axis=1)

    sorted_x = pl.pallas_call(
        _bitonic_sort_kernel,
        grid=(B, 2),
        in_specs=[pl.BlockSpec((1, 1, rows, _LANES), lambda i, j: (i, j, 0, 0))],
        out_specs=pl.BlockSpec((1, 1, rows, _LANES), lambda i, j: (i, j, 0, 0)),
        out_shape=jax.ShapeDtypeStruct((B, 2, rows, _LANES), jnp.float32),
        compiler_params=pltpu.CompilerParams(
            dimension_semantics=("parallel", "parallel")),
    )(x)

    ss = pl.pallas_call(
        _diff_norm_kernel,
        grid=(B,),
        in_specs=[pl.BlockSpec((1, 2, rows, _LANES), lambda i: (i, 0, 0, 0))],
        out_specs=pl.BlockSpec((1, 8, _LANES), lambda i: (i, 0, 0)),
        out_shape=jax.ShapeDtypeStruct((B, 8, _LANES), jnp.float32),
        compiler_params=pltpu.CompilerParams(
            dimension_semantics=("parallel",)),
    )(sorted_x)

    return jnp.mean(jnp.sqrt(ss[:, 0, 0]))


# speedup vs baseline: 5.4563x; 1.0195x over previous
import jax
import jax.numpy as jnp
from jax import lax
from jax.experimental import pallas as pl
from jax.experimental.pallas import tpu as pltpu

_LANES = 128
_VALID_LANES = 96


def _bitonic_sort_kernel(x_ref, o_ref):
    rows = x_ref.shape[2]
    row_log = rows.bit_length() - 1
    n_log = row_log + 7
    chunk_log = min(6, row_log)
    C = 1 << chunk_log
    nchunks = rows // C

    riota = lax.broadcasted_iota(jnp.int32, (C, _LANES), 0)
    liota = lax.broadcasted_iota(jnp.int32, (C, _LANES), 1)

    def up_mask(k_log, c1):
        if k_log < chunk_log:
            return (riota & (1 << k_log)) == 0
        if k_log < row_log:
            return (c1 & (1 << (k_log - chunk_log))) == 0
        return (liota & (1 << (k_log - row_log))) == 0

    def cx_routed(x, up, dr):
        g = x.reshape(C // (2 * dr), 2, dr, _LANES)
        a, b = g[:, 0], g[:, 1]
        m = jnp.minimum(a, b)
        mx = jnp.maximum(a, b)
        if isinstance(up, jax.Array) and up.ndim:
            up = up.reshape(C // (2 * dr), 2, dr, _LANES)[:, 0]
        lo = jnp.where(up, m, mx)
        hi = jnp.where(up, mx, m)
        return jnp.stack([lo, hi], axis=1).reshape(C, _LANES)

    def cx_roll(x, up, axis, size, d, iota):
        is_lo = (iota & d) == 0
        y = pltpu.roll(x, size - d, axis)
        z = pltpu.roll(x, d, axis)
        p = jnp.where(is_lo, y, z)
        m = jnp.minimum(x, p)
        mx = jnp.maximum(x, p)
        return jnp.where(up == is_lo, m, mx)

    def row_chain(x, k_log, c1, start_j):
        up = up_mask(k_log, c1)
        for j in range(start_j, -1, -1):
            d = 1 << j
            if d >= 8:
                x = cx_routed(x, up, d)
            else:
                x = cx_roll(x, up, 0, C, d, riota)
        return x

    def lane_chain(x, k_log, lane_js):
        up = up_mask(k_log, 0)
        for j in lane_js:
            x = cx_roll(x, up, 1, _LANES, 1 << (j - row_log), liota)
        return x

    def rd(c):
        return o_ref[0, 0, pl.ds(c * C, C), :]

    def wr(c, v):
        o_ref[0, 0, pl.ds(c * C, C), :] = v

    def sweep1_one(c):
        x = x_ref[0, 0, pl.ds(c * C, C), :]
        for k_log in range(1, chunk_log + 1):
            x = row_chain(x, k_log, c, k_log - 1)
        wr(c, x)

    if nchunks % 2 == 0:
        def sweep1(g, _):
            sweep1_one(2 * g)
            sweep1_one(2 * g + 1)
            return 0

        lax.fori_loop(0, nchunks // 2, sweep1, 0)
    else:
        def sweep1(c, _):
            sweep1_one(c)
            return 0

        lax.fori_loop(0, nchunks, sweep1, 0)

    for k_log in range(chunk_log + 1, n_log + 1):
        lane_js = list(range(k_log - 1, row_log - 1, -1))
        cross_js = list(range(min(k_log - 1, row_log - 1), chunk_log - 1, -1))

        if not cross_js:
            def solo_sweep(c, _, k_log=k_log, lane_js=lane_js):
                x = rd(c)
                x = lane_chain(x, k_log, lane_js)
                x = row_chain(x, k_log, c, chunk_log - 1)
                wr(c, x)
                return 0

            lax.fori_loop(0, nchunks, solo_sweep, 0)
            continue

        def cross_pairs(j, body_fn):
            s = 1 << (j - chunk_log)
            b = j - chunk_log
            npairs = nchunks // 2

            def one(g):
                c1 = ((g >> b) << (b + 1)) | (g & (s - 1))
                body_fn(c1, c1 + s)

            if npairs % 2 == 0:
                def body(g, _):
                    one(2 * g)
                    one(2 * g + 1)
                    return 0

                lax.fori_loop(0, npairs // 2, body, 0)
            else:
                def body(g, _):
                    one(g)
                    return 0

                lax.fori_loop(0, npairs, body, 0)

        def cross_cx(c1, c2, a, bv, k_log=k_log):
            up = up_mask(k_log, c1)
            m = jnp.minimum(a, bv)
            mx = jnp.maximum(a, bv)
            return jnp.where(up, m, mx), jnp.where(up, mx, m)

        def first_sweep(c1, c2, k_log=k_log, lane_js=lane_js):
            a, bv = rd(c1), rd(c2)
            a = lane_chain(a, k_log, lane_js)
            bv = lane_chain(bv, k_log, lane_js)
            a, bv = cross_cx(c1, c2, a, bv)
            wr(c1, a)
            wr(c2, bv)

        cross_pairs(cross_js[0], first_sweep)

        mids = cross_js[1:-1]
        while mids:
            if len(mids) >= 2:
                j1 = mids[0]
                b = j1 - 1 - chunk_log
                s2 = 1 << b

                def quad_body(g, _, k_log=k_log, b=b, s2=s2):
                    c0 = ((g >> b) << (b + 2)) | (g & (s2 - 1))
                    va = rd(c0)
                    vb = rd(c0 + s2)
                    vc = rd(c0 + 2 * s2)
                    vd = rd(c0 + 3 * s2)
                    va, vc = cross_cx(c0, 0, va, vc)
                    vb, vd = cross_cx(c0, 0, vb, vd)
                    va, vb = cross_cx(c0, 0, va, vb)
                    vc, vd = cross_cx(c0, 0, vc, vd)
                    wr(c0, va)
                    wr(c0 + s2, vb)
                    wr(c0 + 2 * s2, vc)
                    wr(c0 + 3 * s2, vd)
                    return 0

                lax.fori_loop(0, nchunks // 4, quad_body, 0)
                mids = mids[2:]
            else:
                def mid_sweep(c1, c2, k_log=k_log):
                    a, bv = cross_cx(c1, c2, rd(c1), rd(c2))
                    wr(c1, a)
                    wr(c2, bv)

                cross_pairs(mids[0], mid_sweep)
                mids = mids[1:]

        if len(cross_js) > 1:
            def last_sweep(c1, c2, k_log=k_log):
                a, bv = cross_cx(c1, c2, rd(c1), rd(c2))
                a = row_chain(a, k_log, c1, chunk_log - 1)
                bv = row_chain(bv, k_log, c2, chunk_log - 1)
                wr(c1, a)
                wr(c2, bv)

            cross_pairs(chunk_log, last_sweep)
        else:
            def tail_sweep(c, _, k_log=k_log):
                wr(c, row_chain(rd(c), k_log, c, chunk_log - 1))
                return 0

            lax.fori_loop(0, nchunks, tail_sweep, 0)


def _diff_norm_kernel(s_ref, o_ref):
    a = s_ref[0, 0]
    b = s_ref[0, 1]
    lane_iota = lax.broadcasted_iota(jnp.int32, a.shape, 1)
    d = jnp.where(lane_iota < _VALID_LANES, a - b, 0.0)
    o_ref[0] = jnp.full(o_ref.shape[1:], jnp.sum(d * d), jnp.float32)


def kernel(pc1, pc2):
    B = pc1.shape[0]
    n = pc1.shape[1] * pc1.shape[2]
    rows = n // _VALID_LANES
    assert rows * _VALID_LANES == n and rows & (rows - 1) == 0

    def prep(pc):
        v = pc.reshape(B, _VALID_LANES, rows).transpose(0, 2, 1)
        return jnp.pad(v, ((0, 0), (0, 0), (0, _LANES - _VALID_LANES)),
                       constant_values=jnp.inf)

    x = jnp.stack([prep(pc1), prep(pc2)], axis=1)

    sorted_x = pl.pallas_call(
        _bitonic_sort_kernel,
        grid=(B, 2),
        in_specs=[pl.BlockSpec((1, 1, rows, _LANES), lambda i, j: (i, j, 0, 0))],
        out_specs=pl.BlockSpec((1, 1, rows, _LANES), lambda i, j: (i, j, 0, 0)),
        out_shape=jax.ShapeDtypeStruct((B, 2, rows, _LANES), jnp.float32),
        compiler_params=pltpu.CompilerParams(
            dimension_semantics=("parallel", "parallel")),
    )(x)

    ss = pl.pallas_call(
        _diff_norm_kernel,
        grid=(B,),
        in_specs=[pl.BlockSpec((1, 2, rows, _LANES), lambda i: (i, 0, 0, 0))],
        out_specs=pl.BlockSpec((1, 8, _LANES), lambda i: (i, 0, 0)),
        out_shape=jax.ShapeDtypeStruct((B, 8, _LANES), jnp.float32),
        compiler_params=pltpu.CompilerParams(
            dimension_semantics=("parallel",)),
    )(sorted_x)

    return jnp.mean(jnp.sqrt(ss[:, 0, 0]))


# speedup vs baseline: 5.4674x; 1.0020x over previous
import jax
import jax.numpy as jnp
from jax import lax
from jax.experimental import pallas as pl
from jax.experimental.pallas import tpu as pltpu

_LANES = 128
_VALID_LANES = 96


def _bitonic_sort_kernel(x_ref, o_ref):
    rows = x_ref.shape[2]
    row_log = rows.bit_length() - 1
    n_log = row_log + 7
    chunk_log = min(6, row_log)
    C = 1 << chunk_log
    nchunks = rows // C

    riota = lax.broadcasted_iota(jnp.int32, (C, _LANES), 0)
    liota = lax.broadcasted_iota(jnp.int32, (C, _LANES), 1)

    def up_mask(k_log, c1):
        if k_log < chunk_log:
            return (riota & (1 << k_log)) == 0
        if k_log < row_log:
            return (c1 & (1 << (k_log - chunk_log))) == 0
        return (liota & (1 << (k_log - row_log))) == 0

    def cx_routed(x, up, dr):
        g = x.reshape(C // (2 * dr), 2, dr, _LANES)
        a, b = g[:, 0], g[:, 1]
        m = jnp.minimum(a, b)
        mx = jnp.maximum(a, b)
        if isinstance(up, jax.Array) and up.ndim:
            up = up.reshape(C // (2 * dr), 2, dr, _LANES)[:, 0]
        lo = jnp.where(up, m, mx)
        hi = jnp.where(up, mx, m)
        return jnp.stack([lo, hi], axis=1).reshape(C, _LANES)

    def cx_roll(x, up, axis, size, d, iota):
        is_lo = (iota & d) == 0
        y = pltpu.roll(x, size - d, axis)
        z = pltpu.roll(x, d, axis)
        p = jnp.where(is_lo, y, z)
        m = jnp.minimum(x, p)
        mx = jnp.maximum(x, p)
        return jnp.where(up == is_lo, m, mx)

    def row_chain(x, k_log, c1, start_j):
        up = up_mask(k_log, c1)
        for j in range(start_j, -1, -1):
            d = 1 << j
            if d >= 8:
                x = cx_routed(x, up, d)
            else:
                x = cx_roll(x, up, 0, C, d, riota)
        return x

    def lane_chain(x, k_log, lane_js):
        up = up_mask(k_log, 0)
        for j in lane_js:
            x = cx_roll(x, up, 1, _LANES, 1 << (j - row_log), liota)
        return x

    def rd(c):
        return o_ref[0, 0, pl.ds(c * C, C), :]

    def wr(c, v):
        o_ref[0, 0, pl.ds(c * C, C), :] = v

    def chain16(c):
        x = x_ref[0, 0, pl.ds(c * C, C), :]
        for k_log in range(1, chunk_log + 1):
            x = row_chain(x, k_log, c, k_log - 1)
        return x

    if nchunks % 2 == 0:
        k7 = chunk_log + 1

        def sweep1(g, _):
            xa = chain16(2 * g)
            xb = chain16(2 * g + 1)
            up = up_mask(k7, 2 * g)
            m = jnp.minimum(xa, xb)
            mx = jnp.maximum(xa, xb)
            xa = jnp.where(up, m, mx)
            xb = jnp.where(up, mx, m)
            xa = row_chain(xa, k7, 2 * g, chunk_log - 1)
            xb = row_chain(xb, k7, 2 * g + 1, chunk_log - 1)
            wr(2 * g, xa)
            wr(2 * g + 1, xb)
            return 0

        lax.fori_loop(0, nchunks // 2, sweep1, 0)
        start_level = chunk_log + 2
    else:
        def sweep1(c, _):
            wr(c, chain16(c))
            return 0

        lax.fori_loop(0, nchunks, sweep1, 0)
        start_level = chunk_log + 1

    for k_log in range(start_level, n_log + 1):
        lane_js = list(range(k_log - 1, row_log - 1, -1))
        cross_js = list(range(min(k_log - 1, row_log - 1), chunk_log - 1, -1))

        if not cross_js:
            def solo_sweep(c, _, k_log=k_log, lane_js=lane_js):
                x = rd(c)
                x = lane_chain(x, k_log, lane_js)
                x = row_chain(x, k_log, c, chunk_log - 1)
                wr(c, x)
                return 0

            lax.fori_loop(0, nchunks, solo_sweep, 0)
            continue

        def cross_pairs(j, body_fn):
            s = 1 << (j - chunk_log)
            b = j - chunk_log
            npairs = nchunks // 2

            def one(g):
                c1 = ((g >> b) << (b + 1)) | (g & (s - 1))
                body_fn(c1, c1 + s)

            if npairs % 2 == 0:
                def body(g, _):
                    one(2 * g)
                    one(2 * g + 1)
                    return 0

                lax.fori_loop(0, npairs // 2, body, 0)
            else:
                def body(g, _):
                    one(g)
                    return 0

                lax.fori_loop(0, npairs, body, 0)

        def cross_cx(c1, c2, a, bv, k_log=k_log):
            up = up_mask(k_log, c1)
            m = jnp.minimum(a, bv)
            mx = jnp.maximum(a, bv)
            return jnp.where(up, m, mx), jnp.where(up, mx, m)

        def first_sweep(c1, c2, k_log=k_log, lane_js=lane_js):
            a, bv = rd(c1), rd(c2)
            a = lane_chain(a, k_log, lane_js)
            bv = lane_chain(bv, k_log, lane_js)
            a, bv = cross_cx(c1, c2, a, bv)
            wr(c1, a)
            wr(c2, bv)

        cross_pairs(cross_js[0], first_sweep)

        mids = cross_js[1:-1]
        while mids:
            if len(mids) >= 2:
                j1 = mids[0]
                b = j1 - 1 - chunk_log
                s2 = 1 << b

                def quad_body(g, _, k_log=k_log, b=b, s2=s2):
                    c0 = ((g >> b) << (b + 2)) | (g & (s2 - 1))
                    va = rd(c0)
                    vb = rd(c0 + s2)
                    vc = rd(c0 + 2 * s2)
                    vd = rd(c0 + 3 * s2)
                    va, vc = cross_cx(c0, 0, va, vc)
                    vb, vd = cross_cx(c0, 0, vb, vd)
                    va, vb = cross_cx(c0, 0, va, vb)
                    vc, vd = cross_cx(c0, 0, vc, vd)
                    wr(c0, va)
                    wr(c0 + s2, vb)
                    wr(c0 + 2 * s2, vc)
                    wr(c0 + 3 * s2, vd)
                    return 0

                lax.fori_loop(0, nchunks // 4, quad_body, 0)
                mids = mids[2:]
            else:
                def mid_sweep(c1, c2, k_log=k_log):
                    a, bv = cross_cx(c1, c2, rd(c1), rd(c2))
                    wr(c1, a)
                    wr(c2, bv)

                cross_pairs(mids[0], mid_sweep)
                mids = mids[1:]

        if len(cross_js) > 1:
            def last_sweep(c1, c2, k_log=k_log):
                a, bv = cross_cx(c1, c2, rd(c1), rd(c2))
                a = row_chain(a, k_log, c1, chunk_log - 1)
                bv = row_chain(bv, k_log, c2, chunk_log - 1)
                wr(c1, a)
                wr(c2, bv)

            cross_pairs(chunk_log, last_sweep)
        else:
            def tail_sweep(c, _, k_log=k_log):
                wr(c, row_chain(rd(c), k_log, c, chunk_log - 1))
                return 0

            lax.fori_loop(0, nchunks, tail_sweep, 0)


def _diff_norm_kernel(s_ref, o_ref):
    a = s_ref[0, 0]
    b = s_ref[0, 1]
    lane_iota = lax.broadcasted_iota(jnp.int32, a.shape, 1)
    d = jnp.where(lane_iota < _VALID_LANES, a - b, 0.0)
    o_ref[0] = jnp.full(o_ref.shape[1:], jnp.sum(d * d), jnp.float32)


def kernel(pc1, pc2):
    B = pc1.shape[0]
    n = pc1.shape[1] * pc1.shape[2]
    rows = n // _VALID_LANES
    assert rows * _VALID_LANES == n and rows & (rows - 1) == 0

    def prep(pc):
        v = pc.reshape(B, _VALID_LANES, rows).transpose(0, 2, 1)
        return jnp.pad(v, ((0, 0), (0, 0), (0, _LANES - _VALID_LANES)),
                       constant_values=jnp.inf)

    x = jnp.stack([prep(pc1), prep(pc2)], axis=1)

    sorted_x = pl.pallas_call(
        _bitonic_sort_kernel,
        grid=(B, 2),
        in_specs=[pl.BlockSpec((1, 1, rows, _LANES), lambda i, j: (i, j, 0, 0))],
        out_specs=pl.BlockSpec((1, 1, rows, _LANES), lambda i, j: (i, j, 0, 0)),
        out_shape=jax.ShapeDtypeStruct((B, 2, rows, _LANES), jnp.float32),
        compiler_params=pltpu.CompilerParams(
            dimension_semantics=("parallel", "parallel")),
    )(x)

    ss = pl.pallas_call(
        _diff_norm_kernel,
        grid=(B,),
        in_specs=[pl.BlockSpec((1, 2, rows, _LANES), lambda i: (i, 0, 0, 0))],
        out_specs=pl.BlockSpec((1, 8, _LANES), lambda i: (i, 0, 0)),
        out_shape=jax.ShapeDtypeStruct((B, 8, _LANES), jnp.float32),
        compiler_params=pltpu.CompilerParams(
            dimension_semantics=("parallel",)),
    )(sorted_x)

    return jnp.mean(jnp.sqrt(ss[:, 0, 0]))
